# Initial kernel scaffold; baseline (speedup 1.0000x reference)
import functools

import jax
import jax.numpy as jnp
from jax import lax
from jax.experimental import pallas as pl
from jax.experimental.pallas import tpu as pltpu
from jax.experimental.pallas import tpu_sc as plsc

NT = 16
LANES = 16
CHUNK = 80


def _etype_pass(feat_hbm, src_all_hbm, dst_all_hbm, norm_all_hbm, out_hbm,
                sid, src_v, dst_v, norm_v, rows_v, acc, sem,
                n_out, d):
    n_chunks = src_v.shape[0]
    vregs_per_row = d // LANES
    rpt = n_out // NT

    zero = jnp.zeros((LANES,), jnp.float32)

    def zero_body(e, carry):
        for j in range(vregs_per_row):
            rows_v[e, pl.ds(j * LANES, LANES)] = zero
        return carry

    lax.fori_loop(0, CHUNK, zero_body, 0)
    full = rpt // CHUNK
    rem = rpt % CHUNK
    for t in range(full):
        pltpu.sync_copy(rows_v, acc.at[pl.ds(sid * rpt + t * CHUNK, CHUNK)])
    if rem:
        pltpu.sync_copy(rows_v.at[pl.ds(0, rem)],
                        acc.at[pl.ds(sid * rpt + full * CHUNK, rem)])
    plsc.subcore_barrier()

    pltpu.sync_copy(src_all_hbm.at[sid], src_v)
    pltpu.sync_copy(dst_all_hbm.at[sid], dst_v)
    pltpu.sync_copy(norm_all_hbm.at[sid], norm_v)

    def chunk_body(k, carry):
        pltpu.async_copy(feat_hbm.at[src_v.at[k]], rows_v, sem).wait()

        def mul_body(e, c2):
            n = norm_v[k, e]
            for j in range(vregs_per_row):
                sl = pl.ds(j * LANES, LANES)
                rows_v[e, sl] = rows_v[e, sl] * n
            return c2

        lax.fori_loop(0, CHUNK, mul_body, 0)

        pltpu.sync_copy(rows_v, acc.at[dst_v.at[k]], add=True)
        return carry

    lax.fori_loop(0, n_chunks, chunk_body, 0)
    plsc.subcore_barrier()

    pltpu.sync_copy(acc.at[pl.ds(sid * rpt, rpt)],
                    out_hbm.at[pl.ds(sid * rpt, rpt)])


def _build_kernel(n_users, n_items, e, d):
    ept = e // NT
    n_chunks = ept // CHUNK
    n_acc = max(n_users, n_items)
    mesh = plsc.VectorSubcoreMesh(core_axis_name="c", subcore_axis_name="s")

    @functools.partial(
        pl.kernel,
        mesh=mesh,
        out_type=(jax.ShapeDtypeStruct((n_users, d), jnp.float32),
                  jax.ShapeDtypeStruct((n_items, d), jnp.float32)),
        scratch_types=[
            pltpu.VMEM((n_chunks, CHUNK), jnp.int32),
            pltpu.VMEM((n_chunks, CHUNK), jnp.int32),
            pltpu.VMEM((n_chunks, CHUNK), jnp.float32),
            pltpu.VMEM((CHUNK, d), jnp.float32),
            pltpu.VMEM_SHARED((n_acc, d), jnp.float32),
            pltpu.SemaphoreType.DMA,
        ],
    )
    def gcn_kernel(user_feat, item_feat,
                   ui_src, ui_dst, norm_ui,
                   iu_src, iu_dst, norm_iu,
                   user_h, item_h,
                   src_v, dst_v, norm_v, rows_v, acc, sem):
        cid = lax.axis_index("c")
        sid = lax.axis_index("s")

        @pl.when(cid == 0)
        def _():
            _etype_pass(user_feat, ui_src, ui_dst, norm_ui, item_h,
                        sid, src_v, dst_v, norm_v, rows_v, acc, sem,
                        n_items, d)

        @pl.when(cid == 1)
        def _():
            _etype_pass(item_feat, iu_src, iu_dst, norm_iu, user_h,
                        sid, src_v, dst_v, norm_v, rows_v, acc, sem,
                        n_users, d)

    return gcn_kernel


def kernel(user_feat, item_feat, ui_edges, iu_edges, norm_ui, norm_iu):
    n_users, d = user_feat.shape
    n_items = item_feat.shape[0]
    e = ui_edges.shape[1]
    ept = e // NT

    ui = ui_edges.astype(jnp.int32)
    iu = iu_edges.astype(jnp.int32)
    shape3 = (NT, ept // CHUNK, CHUNK)
    fn = _build_kernel(n_users, n_items, e, d)
    user_h, item_h = fn(
        user_feat, item_feat,
        ui[0].reshape(shape3), ui[1].reshape(shape3),
        norm_ui.reshape(shape3),
        iu[0].reshape(shape3), iu[1].reshape(shape3),
        norm_iu.reshape(shape3),
    )
    return (user_h, item_h)


# speedup vs baseline: 1.6169x; 1.6169x over previous
import functools

import jax
import jax.numpy as jnp
from jax import lax
from jax.experimental import pallas as pl
from jax.experimental.pallas import tpu as pltpu
from jax.experimental.pallas import tpu_sc as plsc

NT = 16
LANES = 16
CHUNK = 80
ACC_ROWS = 2560
TRASH = ACC_ROWS


def _zero_rows_v(rows_v, vregs_per_row):
    zero = jnp.zeros((LANES,), jnp.float32)

    def zero_body(e, carry):
        for j in range(vregs_per_row):
            rows_v[e, pl.ds(j * LANES, LANES)] = zero
        return carry

    lax.fori_loop(0, CHUNK, zero_body, 0)


def _copy_row_range(src_ref, src_base, dst_ref, dst_base, nrows):
    done = 0
    while done < nrows:
        blk = min(CHUNK, nrows - done)
        pltpu.sync_copy(src_ref.at[pl.ds(src_base + done, blk)],
                        dst_ref.at[pl.ds(dst_base + done, blk)])
        done += blk


def _etype_pass(feat_hbm, src_all_hbm, dst_all_hbm, norm_all_hbm, out_hbm,
                sid, src_v, dst_v, norm_v, rows_v, dst_buf, acc, sem,
                n_out, d):
    n_chunks = src_v.shape[0]
    vregs_per_row = d // LANES
    groups = CHUNK // LANES
    n_pass = -(-n_out // ACC_ROWS)

    pltpu.sync_copy(src_all_hbm.at[sid], src_v)
    pltpu.sync_copy(dst_all_hbm.at[sid], dst_v)
    pltpu.sync_copy(norm_all_hbm.at[sid], norm_v)

    for p in range(n_pass):
        row0 = p * ACC_ROWS
        rows_p = min(ACC_ROWS, n_out - row0)
        rpt = (rows_p // NT) // 8 * 8
        tail_start = rpt * NT
        tail_rows = rows_p - tail_start

        _zero_rows_v(rows_v, vregs_per_row)
        full = rpt // CHUNK
        rem = rpt % CHUNK
        for t in range(full):
            pltpu.sync_copy(rows_v,
                            acc.at[pl.ds(sid * rpt + t * CHUNK, CHUNK)])
        if rem:
            pltpu.sync_copy(rows_v.at[pl.ds(0, rem)],
                            acc.at[pl.ds(sid * rpt + full * CHUNK, rem)])
        if tail_rows:
            @pl.when(sid == NT - 1)
            def _():
                pltpu.sync_copy(rows_v.at[pl.ds(0, tail_rows)],
                                acc.at[pl.ds(tail_start, tail_rows)])
        plsc.subcore_barrier()

        def chunk_body(k, carry):
            for g in range(groups):
                sl = pl.ds(g * LANES, LANES)
                dv = dst_v[k, sl] - row0
                m = (dv >= 0) & (dv < rows_p)
                dst_buf[sl] = jnp.where(m, dv, TRASH)

            pltpu.async_copy(feat_hbm.at[src_v.at[k]], rows_v, sem).wait()

            def mul_body(g, c2):
                nv = norm_v[k, pl.ds(g * LANES, LANES)]
                for i in range(LANES):
                    e = g * LANES + i
                    n = nv[i]
                    for j in range(vregs_per_row):
                        sl = pl.ds(j * LANES, LANES)
                        rows_v[e, sl] = rows_v[e, sl] * n
                return c2

            lax.fori_loop(0, groups, mul_body, 0)

            pltpu.sync_copy(rows_v, acc.at[dst_buf], add=True)
            return carry

        lax.fori_loop(0, n_chunks, chunk_body, 0)
        plsc.subcore_barrier()

        _copy_row_range(acc, sid * rpt, out_hbm, row0 + sid * rpt, rpt)
        if tail_rows:
            @pl.when(sid == NT - 1)
            def _():
                _copy_row_range(acc, tail_start, out_hbm,
                                row0 + tail_start, tail_rows)


def _build_kernel(n_users, n_items, e, d):
    ept = e // NT
    n_chunks = ept // CHUNK
    mesh = plsc.VectorSubcoreMesh(core_axis_name="c", subcore_axis_name="s")

    @functools.partial(
        pl.kernel,
        mesh=mesh,
        out_type=(jax.ShapeDtypeStruct((n_users, d), jnp.float32),
                  jax.ShapeDtypeStruct((n_items, d), jnp.float32)),
        scratch_types=[
            pltpu.VMEM((n_chunks, CHUNK), jnp.int32),
            pltpu.VMEM((n_chunks, CHUNK), jnp.int32),
            pltpu.VMEM((n_chunks, CHUNK), jnp.float32),
            pltpu.VMEM((CHUNK, d), jnp.float32),
            pltpu.VMEM((CHUNK,), jnp.int32),
            pltpu.VMEM_SHARED((ACC_ROWS + 8, d), jnp.float32),
            pltpu.SemaphoreType.DMA,
        ],
    )
    def gcn_kernel(user_feat, item_feat,
                   ui_src, ui_dst, norm_ui,
                   iu_src, iu_dst, norm_iu,
                   user_h, item_h,
                   src_v, dst_v, norm_v, rows_v, dst_buf, acc, sem):
        cid = lax.axis_index("c")
        sid = lax.axis_index("s")

        @pl.when(cid == 0)
        def _():
            _etype_pass(user_feat, ui_src, ui_dst, norm_ui, item_h,
                        sid, src_v, dst_v, norm_v, rows_v, dst_buf, acc, sem,
                        n_items, d)

        @pl.when(cid == 1)
        def _():
            _etype_pass(item_feat, iu_src, iu_dst, norm_iu, user_h,
                        sid, src_v, dst_v, norm_v, rows_v, dst_buf, acc, sem,
                        n_users, d)

    return gcn_kernel


def kernel(user_feat, item_feat, ui_edges, iu_edges, norm_ui, norm_iu):
    n_users, d = user_feat.shape
    n_items = item_feat.shape[0]
    e = ui_edges.shape[1]
    ept = e // NT

    ui = ui_edges.astype(jnp.int32)
    iu = iu_edges.astype(jnp.int32)
    shape3 = (NT, ept // CHUNK, CHUNK)
    fn = _build_kernel(n_users, n_items, e, d)
    user_h, item_h = fn(
        user_feat, item_feat,
        ui[0].reshape(shape3), ui[1].reshape(shape3),
        norm_ui.reshape(shape3),
        iu[0].reshape(shape3), iu[1].reshape(shape3),
        norm_iu.reshape(shape3),
    )
    return (user_h, item_h)


# speedup vs baseline: 3.0698x; 1.8986x over previous
import functools

import jax
import jax.numpy as jnp
from jax import lax
from jax.experimental import pallas as pl
from jax.experimental.pallas import tpu as pltpu
from jax.experimental.pallas import tpu_sc as plsc

NT = 16
LANES = 16
CHUNK = 80
ACC_ROWS = 5000
TRASH = ACC_ROWS


def _zero_rows_v(rows_v, vregs_per_row):
    zero = jnp.zeros((LANES,), jnp.float32)

    def zero_body(e, carry):
        for j in range(vregs_per_row):
            rows_v[e, pl.ds(j * LANES, LANES)] = zero
        return carry

    lax.fori_loop(0, CHUNK, zero_body, 0)


def _copy_row_range(src_ref, src_base, dst_ref, dst_base, nrows):
    done = 0
    while done < nrows:
        blk = min(CHUNK, nrows - done)
        pltpu.sync_copy(src_ref.at[pl.ds(src_base + done, blk)],
                        dst_ref.at[pl.ds(dst_base + done, blk)])
        done += blk


def _etype_pass(feat_hbm, meta_hbm, norm_hbm, out_hbm, sid,
                rows, mbufs, nbufs, dbufs, acc, semg, semm, semn,
                n_out, d, n_chunks):
    vregs_per_row = d // LANES
    groups = CHUNK // LANES
    n_pass = -(-n_out // ACC_ROWS)
    mbase = sid * n_chunks

    for p in range(n_pass):
        row0 = p * ACC_ROWS
        rows_p = min(ACC_ROWS, n_out - row0)
        rpt = (rows_p // NT) // 8 * 8
        tail_start = rpt * NT
        tail_rows = rows_p - tail_start

        _zero_rows_v(rows[0], vregs_per_row)
        full = rpt // CHUNK
        rem = rpt % CHUNK
        for t in range(full):
            pltpu.sync_copy(rows[0],
                            acc.at[pl.ds(sid * rpt + t * CHUNK, CHUNK)])
        if rem:
            pltpu.sync_copy(rows[0].at[pl.ds(0, rem)],
                            acc.at[pl.ds(sid * rpt + full * CHUNK, rem)])
        if tail_rows:
            @pl.when(sid == NT - 1)
            def _():
                pltpu.sync_copy(rows[0].at[pl.ds(0, tail_rows)],
                                acc.at[pl.ds(tail_start, tail_rows)])
        plsc.subcore_barrier()

        def visit(k, b, do_meta2, do_next):
            rv = rows[b]
            mb = mbufs[b]
            nb_ = nbufs[b]
            db = dbufs[b]
            pltpu.make_async_copy(feat_hbm.at[mb.at[0]], rv, semg[b]).wait()

            for g in range(groups):
                sl = pl.ds(g * LANES, LANES)
                dv = mb[1, sl] - row0
                m = (dv >= 0) & (dv < rows_p)
                db[sl] = jnp.where(m, dv, TRASH)

            def mul_body(g, c2):
                nv = nb_[pl.ds(g * LANES, LANES)]
                for i in range(LANES):
                    e = g * LANES + i
                    n = nv[i]
                    for j in range(vregs_per_row):
                        sl = pl.ds(j * LANES, LANES)
                        rv[e, sl] = rv[e, sl] * n
                return c2

            lax.fori_loop(0, groups, mul_body, 0)

            pltpu.sync_copy(rv, acc.at[db], add=True)

            if do_meta2:
                pltpu.async_copy(meta_hbm.at[mbase + k + 2], mb, semm[b])
                pltpu.async_copy(norm_hbm.at[mbase + k + 2], nb_, semn[b])
            if do_next:
                ob = 1 - b
                pltpu.make_async_copy(meta_hbm.at[mbase + k + 1],
                                      mbufs[ob], semm[ob]).wait()
                pltpu.make_async_copy(norm_hbm.at[mbase + k + 1],
                                      nbufs[ob], semn[ob]).wait()
                pltpu.async_copy(feat_hbm.at[mbufs[ob].at[0]],
                                 rows[ob], semg[ob])

        pltpu.sync_copy(meta_hbm.at[mbase], mbufs[0])
        pltpu.sync_copy(norm_hbm.at[mbase], nbufs[0])
        pltpu.async_copy(feat_hbm.at[mbufs[0].at[0]], rows[0], semg[0])
        pltpu.async_copy(meta_hbm.at[mbase + 1], mbufs[1], semm[1])
        pltpu.async_copy(norm_hbm.at[mbase + 1], nbufs[1], semn[1])

        def chunk_pair(i, carry):
            visit(2 * i, 0, True, True)
            visit(2 * i + 1, 1, True, True)
            return carry

        lax.fori_loop(0, n_chunks // 2 - 1, chunk_pair, 0)
        visit(n_chunks - 2, 0, False, True)
        visit(n_chunks - 1, 1, False, False)
        plsc.subcore_barrier()

        _copy_row_range(acc, sid * rpt, out_hbm, row0 + sid * rpt, rpt)
        if tail_rows:
            @pl.when(sid == NT - 1)
            def _():
                _copy_row_range(acc, tail_start, out_hbm,
                                row0 + tail_start, tail_rows)


def _build_kernel(n_users, n_items, e, d):
    ept = e // NT
    n_chunks = ept // CHUNK
    mesh = plsc.VectorSubcoreMesh(core_axis_name="c", subcore_axis_name="s")

    @functools.partial(
        pl.kernel,
        mesh=mesh,
        out_type=(jax.ShapeDtypeStruct((n_users, d), jnp.float32),
                  jax.ShapeDtypeStruct((n_items, d), jnp.float32)),
        scratch_types=[
            pltpu.VMEM((CHUNK, d), jnp.float32),
            pltpu.VMEM((CHUNK, d), jnp.float32),
            pltpu.VMEM((2, CHUNK), jnp.int32),
            pltpu.VMEM((2, CHUNK), jnp.int32),
            pltpu.VMEM((CHUNK,), jnp.float32),
            pltpu.VMEM((CHUNK,), jnp.float32),
            pltpu.VMEM((CHUNK,), jnp.int32),
            pltpu.VMEM((CHUNK,), jnp.int32),
            pltpu.VMEM_SHARED((ACC_ROWS + 8, d), jnp.float32),
            pltpu.SemaphoreType.DMA,
            pltpu.SemaphoreType.DMA,
            pltpu.SemaphoreType.DMA,
            pltpu.SemaphoreType.DMA,
            pltpu.SemaphoreType.DMA,
            pltpu.SemaphoreType.DMA,
        ],
    )
    def gcn_kernel(user_feat, item_feat, meta_ui, norm_ui2, meta_iu,
                   norm_iu2, user_h, item_h,
                   rows_a, rows_b, meta_a, meta_b, norm_a, norm_b,
                   dst_a, dst_b, acc,
                   semg0, semg1, semm0, semm1, semn0, semn1):
        cid = lax.axis_index("c")
        sid = lax.axis_index("s")
        rows = (rows_a, rows_b)
        mbufs = (meta_a, meta_b)
        nbufs = (norm_a, norm_b)
        dbufs = (dst_a, dst_b)
        semg = (semg0, semg1)
        semm = (semm0, semm1)
        semn = (semn0, semn1)

        @pl.when(cid == 0)
        def _():
            _etype_pass(user_feat, meta_ui, norm_ui2, item_h, sid,
                        rows, mbufs, nbufs, dbufs, acc, semg, semm, semn,
                        n_items, d, n_chunks)

        @pl.when(cid == 1)
        def _():
            _etype_pass(item_feat, meta_iu, norm_iu2, user_h, sid,
                        rows, mbufs, nbufs, dbufs, acc, semg, semm, semn,
                        n_users, d, n_chunks)

    return gcn_kernel


def _pack_meta(edges, n_chunks):
    src = edges[0].reshape(NT, n_chunks, 1, CHUNK)
    dst = edges[1].reshape(NT, n_chunks, 1, CHUNK)
    return jnp.concatenate([src, dst], axis=2).reshape(
        NT * n_chunks, 2, CHUNK)


def kernel(user_feat, item_feat, ui_edges, iu_edges, norm_ui, norm_iu):
    n_users, d = user_feat.shape
    n_items = item_feat.shape[0]
    e = ui_edges.shape[1]
    n_chunks = (e // NT) // CHUNK

    ui = ui_edges.astype(jnp.int32)
    iu = iu_edges.astype(jnp.int32)
    fn = _build_kernel(n_users, n_items, e, d)
    user_h, item_h = fn(
        user_feat, item_feat,
        _pack_meta(ui, n_chunks),
        norm_ui.reshape(NT * n_chunks, CHUNK).astype(jnp.float32),
        _pack_meta(iu, n_chunks),
        norm_iu.reshape(NT * n_chunks, CHUNK).astype(jnp.float32),
    )
    return (user_h, item_h)


# speedup vs baseline: 3.7554x; 1.2233x over previous
import functools

import jax
import jax.numpy as jnp
from jax import lax
from jax.experimental import pallas as pl
from jax.experimental.pallas import tpu as pltpu
from jax.experimental.pallas import tpu_sc as plsc

NT = 16
LANES = 16
CHUNK = 80
ACC_ROWS = 5000
TRASH = ACC_ROWS


def _zero_rows_v(rows_v, vregs_per_row):
    zero = jnp.zeros((LANES,), jnp.float32)

    def zero_body(e, carry):
        for j in range(vregs_per_row):
            rows_v[e, pl.ds(j * LANES, LANES)] = zero
        return carry

    lax.fori_loop(0, CHUNK, zero_body, 0)


def _copy_row_range(src_ref, src_base, dst_ref, dst_base, nrows):
    done = 0
    while done < nrows:
        blk = min(CHUNK, nrows - done)
        pltpu.sync_copy(src_ref.at[pl.ds(src_base + done, blk)],
                        dst_ref.at[pl.ds(dst_base + done, blk)])
        done += blk


def _etype_pass(feat_hbm, meta_hbm, norm_hbm, out_hbm, sid,
                rows, mbufs, nbufs, dbufs, acc, semg, semm, semn,
                semsc, n_out, d, n_chunks):
    vregs_per_row = d // LANES
    groups = CHUNK // LANES
    n_pass = -(-n_out // ACC_ROWS)
    mbase = sid * n_chunks

    for p in range(n_pass):
        row0 = p * ACC_ROWS
        rows_p = min(ACC_ROWS, n_out - row0)
        rpt = (rows_p // NT) // 8 * 8
        tail_start = rpt * NT
        tail_rows = rows_p - tail_start

        _zero_rows_v(rows[0], vregs_per_row)
        full = rpt // CHUNK
        rem = rpt % CHUNK
        for t in range(full):
            pltpu.sync_copy(rows[0],
                            acc.at[pl.ds(sid * rpt + t * CHUNK, CHUNK)])
        if rem:
            pltpu.sync_copy(rows[0].at[pl.ds(0, rem)],
                            acc.at[pl.ds(sid * rpt + full * CHUNK, rem)])
        if tail_rows:
            @pl.when(sid == NT - 1)
            def _():
                pltpu.sync_copy(rows[0].at[pl.ds(0, tail_rows)],
                                acc.at[pl.ds(tail_start, tail_rows)])
        plsc.subcore_barrier()

        def visit(k, b, do_meta2, do_next, do_drain):
            rv = rows[b]
            mb = mbufs[b]
            nb_ = nbufs[b]
            db = dbufs[b]
            pltpu.make_async_copy(feat_hbm.at[mb.at[0]], rv, semg[b]).wait()

            for g in range(groups):
                sl = pl.ds(g * LANES, LANES)
                dv = mb[1, sl] - row0
                m = (dv >= 0) & (dv < rows_p)
                db[sl] = jnp.where(m, dv, TRASH)

            def mul_body(g, c2):
                nv = nb_[pl.ds(g * LANES, LANES)]
                for i in range(LANES):
                    e = g * LANES + i
                    n = nv[i]
                    for j in range(vregs_per_row):
                        sl = pl.ds(j * LANES, LANES)
                        rv[e, sl] = rv[e, sl] * n
                return c2

            lax.fori_loop(0, groups, mul_body, 0)

            pltpu.async_copy(rv, acc.at[db], semsc[b], add=True)

            if do_meta2:
                pltpu.async_copy(meta_hbm.at[mbase + k + 2], mb, semm[b])
                pltpu.async_copy(norm_hbm.at[mbase + k + 2], nb_, semn[b])
            if do_next:
                ob = 1 - b
                if do_drain:
                    pltpu.make_async_copy(rows[ob], acc.at[dbufs[ob]],
                                          semsc[ob]).wait()
                pltpu.make_async_copy(meta_hbm.at[mbase + k + 1],
                                      mbufs[ob], semm[ob]).wait()
                pltpu.make_async_copy(norm_hbm.at[mbase + k + 1],
                                      nbufs[ob], semn[ob]).wait()
                pltpu.async_copy(feat_hbm.at[mbufs[ob].at[0]],
                                 rows[ob], semg[ob])

        pltpu.sync_copy(meta_hbm.at[mbase], mbufs[0])
        pltpu.sync_copy(norm_hbm.at[mbase], nbufs[0])
        pltpu.async_copy(feat_hbm.at[mbufs[0].at[0]], rows[0], semg[0])
        pltpu.async_copy(meta_hbm.at[mbase + 1], mbufs[1], semm[1])
        pltpu.async_copy(norm_hbm.at[mbase + 1], nbufs[1], semn[1])

        visit(0, 0, True, True, False)

        def chunk_pair(i, carry):
            visit(2 * i + 1, 1, True, True, True)
            visit(2 * i + 2, 0, True, True, True)
            return carry

        lax.fori_loop(0, (n_chunks - 4) // 2, chunk_pair, 0)
        visit(n_chunks - 3, 1, True, True, True)
        visit(n_chunks - 2, 0, False, True, True)
        visit(n_chunks - 1, 1, False, False, False)
        pltpu.make_async_copy(rows[0], acc.at[dbufs[0]], semsc[0]).wait()
        pltpu.make_async_copy(rows[1], acc.at[dbufs[1]], semsc[1]).wait()
        plsc.subcore_barrier()

        _copy_row_range(acc, sid * rpt, out_hbm, row0 + sid * rpt, rpt)
        if tail_rows:
            @pl.when(sid == NT - 1)
            def _():
                _copy_row_range(acc, tail_start, out_hbm,
                                row0 + tail_start, tail_rows)


def _build_kernel(n_users, n_items, e, d):
    ept = e // NT
    n_chunks = ept // CHUNK
    mesh = plsc.VectorSubcoreMesh(core_axis_name="c", subcore_axis_name="s")

    @functools.partial(
        pl.kernel,
        mesh=mesh,
        out_type=(jax.ShapeDtypeStruct((n_users, d), jnp.float32),
                  jax.ShapeDtypeStruct((n_items, d), jnp.float32)),
        scratch_types=[
            pltpu.VMEM((CHUNK, d), jnp.float32),
            pltpu.VMEM((CHUNK, d), jnp.float32),
            pltpu.VMEM((2, CHUNK), jnp.int32),
            pltpu.VMEM((2, CHUNK), jnp.int32),
            pltpu.VMEM((CHUNK,), jnp.float32),
            pltpu.VMEM((CHUNK,), jnp.float32),
            pltpu.VMEM((CHUNK,), jnp.int32),
            pltpu.VMEM((CHUNK,), jnp.int32),
            pltpu.VMEM_SHARED((ACC_ROWS + 8, d), jnp.float32),
            pltpu.SemaphoreType.DMA,
            pltpu.SemaphoreType.DMA,
            pltpu.SemaphoreType.DMA,
            pltpu.SemaphoreType.DMA,
            pltpu.SemaphoreType.DMA,
            pltpu.SemaphoreType.DMA,
            pltpu.SemaphoreType.DMA,
            pltpu.SemaphoreType.DMA,
        ],
    )
    def gcn_kernel(user_feat, item_feat, meta_ui, norm_ui2, meta_iu,
                   norm_iu2, user_h, item_h,
                   rows_a, rows_b, meta_a, meta_b, norm_a, norm_b,
                   dst_a, dst_b, acc,
                   semg0, semg1, semm0, semm1, semn0, semn1,
                   semsc0, semsc1):
        cid = lax.axis_index("c")
        sid = lax.axis_index("s")
        rows = (rows_a, rows_b)
        mbufs = (meta_a, meta_b)
        nbufs = (norm_a, norm_b)
        dbufs = (dst_a, dst_b)
        semg = (semg0, semg1)
        semm = (semm0, semm1)
        semn = (semn0, semn1)
        semsc = (semsc0, semsc1)

        @pl.when(cid == 0)
        def _():
            _etype_pass(user_feat, meta_ui, norm_ui2, item_h, sid,
                        rows, mbufs, nbufs, dbufs, acc, semg, semm, semn,
                        semsc, n_items, d, n_chunks)

        @pl.when(cid == 1)
        def _():
            _etype_pass(item_feat, meta_iu, norm_iu2, user_h, sid,
                        rows, mbufs, nbufs, dbufs, acc, semg, semm, semn,
                        semsc, n_users, d, n_chunks)

    return gcn_kernel


def _pack_meta(edges, n_chunks):
    src = edges[0].reshape(NT, n_chunks, 1, CHUNK)
    dst = edges[1].reshape(NT, n_chunks, 1, CHUNK)
    return jnp.concatenate([src, dst], axis=2).reshape(
        NT * n_chunks, 2, CHUNK)


def kernel(user_feat, item_feat, ui_edges, iu_edges, norm_ui, norm_iu):
    n_users, d = user_feat.shape
    n_items = item_feat.shape[0]
    e = ui_edges.shape[1]
    n_chunks = (e // NT) // CHUNK

    ui = ui_edges.astype(jnp.int32)
    iu = iu_edges.astype(jnp.int32)
    fn = _build_kernel(n_users, n_items, e, d)
    user_h, item_h = fn(
        user_feat, item_feat,
        _pack_meta(ui, n_chunks),
        norm_ui.reshape(NT * n_chunks, CHUNK).astype(jnp.float32),
        _pack_meta(iu, n_chunks),
        norm_iu.reshape(NT * n_chunks, CHUNK).astype(jnp.float32),
    )
    return (user_h, item_h)
